# baseline (speedup 1.0000x reference)
import functools

import jax
import jax.numpy as jnp
from jax import lax
from jax.experimental import pallas as pl
from jax.experimental.pallas import tpu as pltpu
from jax.experimental.pallas import tpu_sc as plsc

N_NODES = 10000
D_IN = 128
HIDDEN = 64
N_EDGES = 320000

NC = 2
NS = 16
NW = NC * NS
CHUNK = 128
NBUF = 4
CH_PER_TILE = 80
E_PAD = NW * CH_PER_TILE * CHUNK
ROWS_PER_TILE = 632
ACC_ROWS = NS * ROWS_PER_TILE
ZW = 72


def _proj_body(x_ref, w_ref, c_ref, z_ref):
    z_ref[...] = lax.dot_general(
        x_ref[...], w_ref[...], (((1,), (0,)), ((), ())),
        preferred_element_type=jnp.float32) + c_ref[...]


def _post_body(acc_ref, x_ref, wr_ref, bl_ref, wh_ref, bh_ref, y_ref):
    agg = acc_ref[0, :N_NODES, :HIDDEN] + acc_ref[1, :N_NODES, :HIDDEN]
    cnt = (acc_ref[0, :N_NODES, HIDDEN:HIDDEN + 1]
           + acc_ref[1, :N_NODES, HIDDEN:HIDDEN + 1])
    cnt = jnp.maximum(cnt, 1.0)
    xr = lax.dot_general(
        x_ref[...], wr_ref[...], (((1,), (1,)), ((), ())),
        preferred_element_type=jnp.float32)
    conv = agg / cnt + bl_ref[...] + xr
    h = jnp.maximum(conv, 0.0)
    y = lax.dot_general(
        h, wh_ref[...], (((1,), (0,)), ((), ())),
        preferred_element_type=jnp.float32)
    y_ref[...] = y + bh_ref[0, 0]


def _sc_body(z_hbm, src_hbm, dst_hbm, zrow_hbm,
             acc_out,
             src_v, dst_v, rows_a, rows_b, rows_c, rows_d,
             acc_sh, sem_a, sem_b, sem_c, sem_d):
    cid = lax.axis_index("c")
    sid = lax.axis_index("s")
    wid = sid * NC + cid
    base = sid * ROWS_PER_TILE

    pltpu.sync_copy(src_hbm.at[wid], src_v)
    pltpu.sync_copy(dst_hbm.at[wid], dst_v)
    pltpu.sync_copy(zrow_hbm, acc_sh.at[pl.ds(base, ROWS_PER_TILE)])
    plsc.subcore_barrier()

    bufs = [rows_a, rows_b, rows_c, rows_d]
    sems = [sem_a, sem_b, sem_c, sem_d]
    for b in range(NBUF):
        pltpu.async_copy(z_hbm.at[src_v.at[b]], bufs[b], sems[b])
    outer = CH_PER_TILE // NBUF

    def body(o, carry):
        for b in range(NBUF):
            j = o * NBUF + b
            pltpu.make_async_copy(z_hbm.at[src_v.at[j]], bufs[b],
                                  sems[b]).wait()
            pltpu.sync_copy(bufs[b], acc_sh.at[dst_v.at[j]], add=True)

            @pl.when(o < outer - 1)
            def _():
                pltpu.async_copy(z_hbm.at[src_v.at[j + NBUF]], bufs[b],
                                 sems[b])
        return carry

    lax.fori_loop(0, outer, body, 0)
    plsc.subcore_barrier()

    pltpu.sync_copy(acc_sh.at[pl.ds(base, ROWS_PER_TILE)],
                    acc_out.at[cid, pl.ds(base, ROWS_PER_TILE)])


_sc_segment_sum = functools.partial(
    pl.kernel,
    out_type=jax.ShapeDtypeStruct((NC, ACC_ROWS, ZW), jnp.float32),
    mesh=plsc.VectorSubcoreMesh(core_axis_name="c", subcore_axis_name="s"),
    compiler_params=pltpu.CompilerParams(use_tc_tiling_on_sc=False),
    scratch_types=[
        pltpu.VMEM((CH_PER_TILE, CHUNK), jnp.int32),
        pltpu.VMEM((CH_PER_TILE, CHUNK), jnp.int32),
        pltpu.VMEM((CHUNK, ZW), jnp.float32),
        pltpu.VMEM((CHUNK, ZW), jnp.float32),
        pltpu.VMEM((CHUNK, ZW), jnp.float32),
        pltpu.VMEM((CHUNK, ZW), jnp.float32),
        pltpu.VMEM_SHARED((ACC_ROWS, ZW), jnp.float32),
        pltpu.SemaphoreType.DMA,
        pltpu.SemaphoreType.DMA,
        pltpu.SemaphoreType.DMA,
        pltpu.SemaphoreType.DMA,
    ],
)(_sc_body)


@jax.jit
def kernel(x, edge_index, W_l, b_l, W_r, W_head, b_head):
    src = edge_index[0].astype(jnp.int32)
    dst = edge_index[1].astype(jnp.int32)
    pad = E_PAD - N_EDGES
    src_p = jnp.concatenate([src, jnp.zeros((pad,), jnp.int32)])
    dst_p = jnp.concatenate([dst, jnp.full((pad,), N_NODES, jnp.int32)])
    src_r = src_p.reshape(NW, CH_PER_TILE, CHUNK)
    dst_r = dst_p.reshape(NW, CH_PER_TILE, CHUNK)

    w_ext = jnp.zeros((D_IN, ZW), jnp.float32).at[:, :HIDDEN].set(W_l.T)
    onehot = jnp.zeros((1, ZW), jnp.float32).at[0, HIDDEN].set(1.0)
    z = pl.pallas_call(
        _proj_body,
        out_shape=jax.ShapeDtypeStruct((N_NODES, ZW), jnp.float32),
    )(x, w_ext, onehot)

    zrow = jnp.zeros((ROWS_PER_TILE, ZW), jnp.float32)
    acc = _sc_segment_sum(z, src_r, dst_r, zrow)

    y = pl.pallas_call(
        _post_body,
        in_specs=[
            pl.BlockSpec(memory_space=pltpu.VMEM),
            pl.BlockSpec(memory_space=pltpu.VMEM),
            pl.BlockSpec(memory_space=pltpu.VMEM),
            pl.BlockSpec(memory_space=pltpu.VMEM),
            pl.BlockSpec(memory_space=pltpu.VMEM),
            pl.BlockSpec(memory_space=pltpu.SMEM),
        ],
        out_shape=jax.ShapeDtypeStruct((N_NODES, 1), jnp.float32),
    )(acc, x, W_r, b_l.reshape(1, HIDDEN), W_head.reshape(HIDDEN, 1),
      b_head.reshape(1, 1))
    return jnp.squeeze(y, axis=-1)


# speedup vs baseline: 1.3235x; 1.3235x over previous
import functools

import jax
import jax.numpy as jnp
from jax import lax
from jax.experimental import pallas as pl
from jax.experimental.pallas import tpu as pltpu
from jax.experimental.pallas import tpu_sc as plsc

N_NODES = 10000
D_IN = 128
HIDDEN = 64
N_EDGES = 320000

NC = 2
NS = 16
NW = NC * NS
CHUNK = 128
NBUF = 4
CH_PER_TILE = 80
E_PAD = NW * CH_PER_TILE * CHUNK
ROWS_PER_TILE = 632
ACC_ROWS = NS * ROWS_PER_TILE
CNT_W = 16


def _proj_body(x_ref, w_ref, z_ref):
    z_ref[...] = lax.dot_general(
        x_ref[...], w_ref[...], (((1,), (1,)), ((), ())),
        preferred_element_type=jnp.float32)


def _post_body(acc_ref, cnt_ref, x_ref, wr_ref, bl_ref, wh_ref, bh_ref, y_ref):
    agg = acc_ref[0, :N_NODES, :] + acc_ref[1, :N_NODES, :]
    cnt = cnt_ref[0, :N_NODES, 0:1] + cnt_ref[1, :N_NODES, 0:1]
    cnt = jnp.maximum(cnt, 1.0)
    xr = lax.dot_general(
        x_ref[...], wr_ref[...], (((1,), (1,)), ((), ())),
        preferred_element_type=jnp.float32)
    conv = agg / cnt + bl_ref[...] + xr
    h = jnp.maximum(conv, 0.0)
    y = lax.dot_general(
        h, wh_ref[...], (((1,), (0,)), ((), ())),
        preferred_element_type=jnp.float32)
    y_ref[...] = y + bh_ref[0, 0]


def _sc_body(z_hbm, src_hbm, dst_hbm, zrow_hbm, zcnt_hbm, ones_hbm,
             acc_out, cnt_out,
             src_v, dst_v, rows_a, rows_b, rows_c, rows_d, ones_v,
             acc_sh, cnt_sh,
             sem_ga, sem_gb, sem_gc, sem_gd,
             sem_sa, sem_sb, sem_sc, sem_sd, sem_c):
    cid = lax.axis_index("c")
    sid = lax.axis_index("s")
    wid = sid * NC + cid
    base = sid * ROWS_PER_TILE

    pltpu.sync_copy(src_hbm.at[wid], src_v)
    pltpu.sync_copy(dst_hbm.at[wid], dst_v)
    pltpu.sync_copy(ones_hbm, ones_v)
    pltpu.sync_copy(zrow_hbm, acc_sh.at[pl.ds(base, ROWS_PER_TILE)])
    pltpu.sync_copy(zcnt_hbm, cnt_sh.at[pl.ds(base, ROWS_PER_TILE)])
    plsc.subcore_barrier()

    bufs = [rows_a, rows_b, rows_c, rows_d]
    gsems = [sem_ga, sem_gb, sem_gc, sem_gd]
    ssems = [sem_sa, sem_sb, sem_sc, sem_sd]
    for b in range(NBUF):
        pltpu.async_copy(z_hbm.at[src_v.at[b]], bufs[b], gsems[b])
    outer = CH_PER_TILE // NBUF

    def body(o, carry):
        for b in range(NBUF):
            j = o * NBUF + b
            pltpu.make_async_copy(z_hbm.at[src_v.at[j]], bufs[b],
                                  gsems[b]).wait()
            pltpu.async_copy(bufs[b], acc_sh.at[dst_v.at[j]], ssems[b],
                             add=True)
            pltpu.async_copy(ones_v, cnt_sh.at[dst_v.at[j]], sem_c,
                             add=True)
        @pl.when(o < outer - 1)
        def _():
            for b in range(NBUF):
                j = o * NBUF + b
                pltpu.make_async_copy(bufs[b], acc_sh.at[dst_v.at[j]],
                                      ssems[b]).wait()
                pltpu.async_copy(z_hbm.at[src_v.at[j + NBUF]], bufs[b],
                                 gsems[b])
        return carry

    lax.fori_loop(0, outer, body, 0)

    for b in range(NBUF):
        j = CH_PER_TILE - NBUF + b
        pltpu.make_async_copy(bufs[b], acc_sh.at[dst_v.at[j]],
                              ssems[b]).wait()

    def drain(j, carry):
        pltpu.make_async_copy(ones_v, cnt_sh.at[dst_v.at[j]], sem_c).wait()
        return carry

    lax.fori_loop(0, CH_PER_TILE, drain, 0)
    plsc.subcore_barrier()

    pltpu.sync_copy(acc_sh.at[pl.ds(base, ROWS_PER_TILE)],
                    acc_out.at[cid, pl.ds(base, ROWS_PER_TILE)])
    pltpu.sync_copy(cnt_sh.at[pl.ds(base, ROWS_PER_TILE)],
                    cnt_out.at[cid, pl.ds(base, ROWS_PER_TILE)])


_sc_segment_sum = functools.partial(
    pl.kernel,
    out_type=(
        jax.ShapeDtypeStruct((NC, ACC_ROWS, HIDDEN), jnp.float32),
        jax.ShapeDtypeStruct((NC, ACC_ROWS, CNT_W), jnp.float32),
    ),
    mesh=plsc.VectorSubcoreMesh(core_axis_name="c", subcore_axis_name="s"),
    compiler_params=pltpu.CompilerParams(use_tc_tiling_on_sc=False),
    scratch_types=[
        pltpu.VMEM((CH_PER_TILE, CHUNK), jnp.int32),
        pltpu.VMEM((CH_PER_TILE, CHUNK), jnp.int32),
        pltpu.VMEM((CHUNK, HIDDEN), jnp.float32),
        pltpu.VMEM((CHUNK, HIDDEN), jnp.float32),
        pltpu.VMEM((CHUNK, HIDDEN), jnp.float32),
        pltpu.VMEM((CHUNK, HIDDEN), jnp.float32),
        pltpu.VMEM((CHUNK, CNT_W), jnp.float32),
        pltpu.VMEM_SHARED((ACC_ROWS, HIDDEN), jnp.float32),
        pltpu.VMEM_SHARED((ACC_ROWS, CNT_W), jnp.float32),
        pltpu.SemaphoreType.DMA,
        pltpu.SemaphoreType.DMA,
        pltpu.SemaphoreType.DMA,
        pltpu.SemaphoreType.DMA,
        pltpu.SemaphoreType.DMA,
        pltpu.SemaphoreType.DMA,
        pltpu.SemaphoreType.DMA,
        pltpu.SemaphoreType.DMA,
        pltpu.SemaphoreType.DMA,
    ],
)(_sc_body)


@jax.jit
def kernel(x, edge_index, W_l, b_l, W_r, W_head, b_head):
    src = edge_index[0].astype(jnp.int32)
    dst = edge_index[1].astype(jnp.int32)
    pad = E_PAD - N_EDGES
    src_p = jnp.concatenate([src, jnp.zeros((pad,), jnp.int32)])
    dst_p = jnp.concatenate([dst, jnp.full((pad,), N_NODES, jnp.int32)])
    src_r = src_p.reshape(NW, CH_PER_TILE, CHUNK)
    dst_r = dst_p.reshape(NW, CH_PER_TILE, CHUNK)

    z = pl.pallas_call(
        _proj_body,
        out_shape=jax.ShapeDtypeStruct((N_NODES, HIDDEN), jnp.float32),
    )(x, W_l)

    zrow = jnp.zeros((ROWS_PER_TILE, HIDDEN), jnp.float32)
    zcnt = jnp.zeros((ROWS_PER_TILE, CNT_W), jnp.float32)
    ones = jnp.ones((CHUNK, CNT_W), jnp.float32)
    acc, cnt = _sc_segment_sum(z, src_r, dst_r, zrow, zcnt, ones)

    y = pl.pallas_call(
        _post_body,
        in_specs=[
            pl.BlockSpec(memory_space=pltpu.VMEM),
            pl.BlockSpec(memory_space=pltpu.VMEM),
            pl.BlockSpec(memory_space=pltpu.VMEM),
            pl.BlockSpec(memory_space=pltpu.VMEM),
            pl.BlockSpec(memory_space=pltpu.VMEM),
            pl.BlockSpec(memory_space=pltpu.VMEM),
            pl.BlockSpec(memory_space=pltpu.SMEM),
        ],
        out_shape=jax.ShapeDtypeStruct((N_NODES, 1), jnp.float32),
    )(acc, cnt, x, W_r, b_l.reshape(1, HIDDEN), W_head.reshape(HIDDEN, 1),
      b_head.reshape(1, 1))
    return jnp.squeeze(y, axis=-1)


# speedup vs baseline: 1.7640x; 1.3329x over previous
import functools

import jax
import jax.numpy as jnp
import numpy as np
from jax import lax
from jax.experimental import pallas as pl
from jax.experimental.pallas import tpu as pltpu
from jax.experimental.pallas import tpu_sc as plsc

N_NODES = 10000
D_IN = 128
HIDDEN = 64
N_EDGES = 320000

NC = 2
NS = 16
NW = NC * NS
CHUNK = 128
NBUF = 2
CH_PER_TILE = 80
N_CHUNKS = NW * CH_PER_TILE
E_PAD = N_CHUNKS * CHUNK
ROWS_PER_TILE = 632
ACC_ROWS = NS * ROWS_PER_TILE
CNT_W = 4
ZPK = HIDDEN // 2
Z_ROWS = N_NODES
Z_RPT = Z_ROWS // NS

_SIGMA = np.concatenate([
    np.arange(16) * 2,
    np.arange(16) * 2 + 1,
    np.arange(16) * 2 + 32,
    np.arange(16) * 2 + 33,
])
_INV_SIGMA = np.argsort(_SIGMA)


def _proj_body(x_ref, w_ref, z_ref):
    z_ref[...] = lax.dot_general(
        x_ref[...], w_ref[...], (((1,), (1,)), ((), ())),
        preferred_element_type=jnp.float32).astype(jnp.bfloat16)


def _post_body(acc_ref, cnt_ref, x_ref, wr_ref, bl_ref, wh_ref, bh_ref, y_ref):
    agg = acc_ref[0, :N_NODES, :] + acc_ref[1, :N_NODES, :]
    cnt_t = jnp.transpose(cnt_ref[...])[:N_NODES]
    cnt = jnp.sum(cnt_t, axis=1, keepdims=True)
    cnt = jnp.maximum(cnt, 1.0)
    xr = lax.dot_general(
        x_ref[...], wr_ref[...], (((1,), (1,)), ((), ())),
        preferred_element_type=jnp.float32)
    conv = agg / cnt + bl_ref[...] + xr
    h = jnp.maximum(conv, 0.0)
    y = lax.dot_general(
        h, wh_ref[...], (((1,), (0,)), ((), ())),
        preferred_element_type=jnp.float32)
    y_ref[...] = y + bh_ref[0, 0]


def _sc_body(z_hbm, src_hbm, dst_hbm, zrow_hbm, zcnt_hbm,
             acc_out, cnt_out,
             src_v, dst_v, ga, gb, fa, fb,
             cnt_tile, z_sh, acc_sh,
             sem_ga, sem_gb, sem_sa, sem_sb):
    cid = lax.axis_index("c")
    sid = lax.axis_index("s")
    wid = sid * NC + cid
    base = sid * ROWS_PER_TILE

    gbufs = [ga, gb]
    fbufs = [fa, fb]
    gsems = [sem_ga, sem_gb]
    ssems = [sem_sa, sem_sb]

    pltpu.sync_copy(z_hbm.at[pl.ds(sid * Z_RPT, Z_RPT)],
                    z_sh.at[pl.ds(sid * Z_RPT, Z_RPT)])
    pltpu.sync_copy(zrow_hbm, acc_sh.at[pl.ds(base, ROWS_PER_TILE)])
    pltpu.sync_copy(zcnt_hbm, cnt_tile)
    pltpu.sync_copy(src_hbm.at[wid], src_v)
    pltpu.sync_copy(dst_hbm.at[wid], dst_v)
    plsc.subcore_barrier()

    for b in range(NBUF):
        pltpu.async_copy(z_sh.at[src_v.at[b]], gbufs[b], gsems[b])
    outer = CH_PER_TILE // NBUF
    himask = jnp.int32(-65536)

    def convert(gbuf, fbuf):
        def rows(r4, carry):
            for dr in range(4):
                r = r4 * 4 + dr
                for h in range(2):
                    w = gbuf[r, pl.ds(h * 16, 16)]
                    lo = plsc.bitcast(w << 16, jnp.float32)
                    hi = plsc.bitcast(w & himask, jnp.float32)
                    fbuf[r, pl.ds(h * 32, 16)] = lo
                    fbuf[r, pl.ds(h * 32 + 16, 16)] = hi
            return carry
        lax.fori_loop(0, CHUNK // 4, rows, 0)

    def body(o, carry):
        for b in range(NBUF):
            j = o * NBUF + b
            @pl.when(o > 0)
            def _():
                pltpu.make_async_copy(fbufs[b], acc_sh.at[dst_v.at[j]],
                                      ssems[b]).wait()
            pltpu.make_async_copy(z_sh.at[src_v.at[j]], gbufs[b],
                                  gsems[b]).wait()
            convert(gbufs[b], fbufs[b])
            @pl.when(o < outer - 1)
            def _():
                pltpu.async_copy(z_sh.at[src_v.at[j + NBUF]], gbufs[b],
                                 gsems[b])
            pltpu.async_copy(fbufs[b], acc_sh.at[dst_v.at[j]], ssems[b],
                             add=True)
            for k in range(CHUNK // 16):
                idx = dst_v[j, pl.ds(k * 16, 16)]
                plsc.addupdate_scatter(
                    cnt_tile, [idx], jnp.full((16,), 1.0, jnp.float32))
        return carry

    lax.fori_loop(0, outer, body, 0)

    for b in range(NBUF):
        j = CH_PER_TILE - NBUF + b
        pltpu.make_async_copy(fbufs[b], acc_sh.at[dst_v.at[j]],
                              ssems[b]).wait()
    plsc.subcore_barrier()

    pltpu.sync_copy(acc_sh.at[pl.ds(base, ROWS_PER_TILE)],
                    acc_out.at[cid, pl.ds(base, ROWS_PER_TILE)])
    pltpu.sync_copy(cnt_tile, cnt_out.at[wid])


_sc_segment_sum = functools.partial(
    pl.kernel,
    out_type=(
        jax.ShapeDtypeStruct((NC, ACC_ROWS, HIDDEN), jnp.float32),
        jax.ShapeDtypeStruct((NW, ACC_ROWS), jnp.float32),
    ),
    mesh=plsc.VectorSubcoreMesh(core_axis_name="c", subcore_axis_name="s"),
    compiler_params=pltpu.CompilerParams(
        use_tc_tiling_on_sc=False, needs_layout_passes=False),
    scratch_types=[
        pltpu.VMEM((CH_PER_TILE, CHUNK), jnp.int32),
        pltpu.VMEM((CH_PER_TILE, CHUNK), jnp.int32),
        pltpu.VMEM((CHUNK, ZPK), jnp.int32),
        pltpu.VMEM((CHUNK, ZPK), jnp.int32),
        pltpu.VMEM((CHUNK, HIDDEN), jnp.float32),
        pltpu.VMEM((CHUNK, HIDDEN), jnp.float32),
        pltpu.VMEM((ACC_ROWS,), jnp.float32),
        pltpu.VMEM_SHARED((Z_ROWS, ZPK), jnp.int32),
        pltpu.VMEM_SHARED((ACC_ROWS, HIDDEN), jnp.float32),
        pltpu.SemaphoreType.DMA,
        pltpu.SemaphoreType.DMA,
        pltpu.SemaphoreType.DMA,
        pltpu.SemaphoreType.DMA,
    ],
)(_sc_body)


@jax.jit
def kernel(x, edge_index, W_l, b_l, W_r, W_head, b_head):
    src = edge_index[0].astype(jnp.int32)
    dst = edge_index[1].astype(jnp.int32)
    pad = E_PAD - N_EDGES
    src_p = jnp.concatenate([src, jnp.zeros((pad,), jnp.int32)])
    dst_p = jnp.concatenate([dst, jnp.full((pad,), N_NODES, jnp.int32)])
    src_r = src_p.reshape(NW, CH_PER_TILE, CHUNK)
    dst_r = dst_p.reshape(NW, CH_PER_TILE, CHUNK)

    w_perm = W_l[jnp.asarray(_INV_SIGMA)]
    z16 = pl.pallas_call(
        _proj_body,
        out_shape=jax.ShapeDtypeStruct((Z_ROWS, HIDDEN), jnp.bfloat16),
    )(x, w_perm)
    z_packed = lax.bitcast_convert_type(
        z16.reshape(Z_ROWS, ZPK, 2), jnp.int32)

    zrow = jnp.zeros((ROWS_PER_TILE, HIDDEN), jnp.float32)
    zcnt = jnp.zeros((ACC_ROWS,), jnp.float32)
    acc, cnt = _sc_segment_sum(z_packed, src_r, dst_r, zrow, zcnt)

    y = pl.pallas_call(
        _post_body,
        in_specs=[
            pl.BlockSpec(memory_space=pltpu.VMEM),
            pl.BlockSpec(memory_space=pltpu.VMEM),
            pl.BlockSpec(memory_space=pltpu.VMEM),
            pl.BlockSpec(memory_space=pltpu.VMEM),
            pl.BlockSpec(memory_space=pltpu.VMEM),
            pl.BlockSpec(memory_space=pltpu.VMEM),
            pl.BlockSpec(memory_space=pltpu.SMEM),
        ],
        out_shape=jax.ShapeDtypeStruct((N_NODES, 1), jnp.float32),
    )(acc, cnt, x, W_r, b_l.reshape(1, HIDDEN), W_head.reshape(HIDDEN, 1),
      b_head.reshape(1, 1))
    return jnp.squeeze(y, axis=-1)


# speedup vs baseline: 1.7661x; 1.0012x over previous
import functools

import jax
import jax.numpy as jnp
import numpy as np
from jax import lax
from jax.experimental import pallas as pl
from jax.experimental.pallas import tpu as pltpu
from jax.experimental.pallas import tpu_sc as plsc

N_NODES = 10000
D_IN = 128
HIDDEN = 64
N_EDGES = 320000

NC = 2
NS = 16
NW = NC * NS
CHUNK = 128
NBUF = 2
CH_PER_TILE = 80
N_CHUNKS = NW * CH_PER_TILE
E_PAD = N_CHUNKS * CHUNK
ROWS_PER_TILE = 632
ACC_ROWS = NS * ROWS_PER_TILE
CNT_W = 4
ZPK = HIDDEN // 2
Z_ROWS = N_NODES
Z_RPT = Z_ROWS // NS

_SIGMA = np.concatenate([
    np.arange(16) * 2,
    np.arange(16) * 2 + 1,
    np.arange(16) * 2 + 32,
    np.arange(16) * 2 + 33,
])
_INV_SIGMA = np.argsort(_SIGMA)


def _proj_body(x_ref, w_ref, z_ref):
    z_ref[...] = lax.dot_general(
        x_ref[...], w_ref[...], (((1,), (1,)), ((), ())),
        preferred_element_type=jnp.float32).astype(jnp.bfloat16)


def _post_body(acc_ref, cnt_ref, x_ref, wr_ref, bl_ref, wh_ref, bh_ref, y_ref):
    agg = acc_ref[0, :N_NODES, :] + acc_ref[1, :N_NODES, :]
    cnt_row = jnp.sum(cnt_ref[...], axis=0, keepdims=True)
    cnt = jnp.transpose(cnt_row)[:N_NODES]
    cnt = jnp.maximum(cnt, 1.0)
    xr = lax.dot_general(
        x_ref[...], wr_ref[...], (((1,), (1,)), ((), ())),
        preferred_element_type=jnp.float32)
    conv = agg / cnt + bl_ref[...] + xr
    h = jnp.maximum(conv, 0.0)
    y = lax.dot_general(
        h, wh_ref[...], (((1,), (0,)), ((), ())),
        preferred_element_type=jnp.float32)
    y_ref[...] = y + bh_ref[0, 0]


def _sc_body(z_hbm, src_hbm, dst_hbm, zrow_hbm, zcnt_hbm,
             acc_out, cnt_out,
             src_v, dst_v, ga, gb, fa, fb,
             cnt_tile, z_sh, acc_sh,
             sem_ga, sem_gb, sem_sa, sem_sb):
    cid = lax.axis_index("c")
    sid = lax.axis_index("s")
    wid = sid * NC + cid
    base = sid * ROWS_PER_TILE

    gbufs = [ga, gb]
    fbufs = [fa, fb]
    gsems = [sem_ga, sem_gb]
    ssems = [sem_sa, sem_sb]

    pltpu.sync_copy(z_hbm.at[pl.ds(sid * Z_RPT, Z_RPT)],
                    z_sh.at[pl.ds(sid * Z_RPT, Z_RPT)])
    pltpu.sync_copy(zrow_hbm, acc_sh.at[pl.ds(base, ROWS_PER_TILE)])
    pltpu.sync_copy(zcnt_hbm, cnt_tile)
    pltpu.sync_copy(src_hbm.at[wid], src_v)
    pltpu.sync_copy(dst_hbm.at[wid], dst_v)
    plsc.subcore_barrier()

    for b in range(NBUF):
        pltpu.async_copy(z_sh.at[src_v.at[b]], gbufs[b], gsems[b])
    outer = CH_PER_TILE // NBUF
    himask = jnp.int32(-65536)

    def convert(gbuf, fbuf):
        def rows(r4, carry):
            for dr in range(4):
                r = r4 * 4 + dr
                for h in range(2):
                    w = gbuf[r, pl.ds(h * 16, 16)]
                    lo = plsc.bitcast(w << 16, jnp.float32)
                    hi = plsc.bitcast(w & himask, jnp.float32)
                    fbuf[r, pl.ds(h * 32, 16)] = lo
                    fbuf[r, pl.ds(h * 32 + 16, 16)] = hi
            return carry
        lax.fori_loop(0, CHUNK // 4, rows, 0)

    def body(o, carry):
        for b in range(NBUF):
            j = o * NBUF + b
            @pl.when(o > 0)
            def _():
                pltpu.make_async_copy(fbufs[b], acc_sh.at[dst_v.at[j]],
                                      ssems[b]).wait()
            pltpu.make_async_copy(z_sh.at[src_v.at[j]], gbufs[b],
                                  gsems[b]).wait()
            convert(gbufs[b], fbufs[b])
            @pl.when(o < outer - 1)
            def _():
                pltpu.async_copy(z_sh.at[src_v.at[j + NBUF]], gbufs[b],
                                 gsems[b])
            pltpu.async_copy(fbufs[b], acc_sh.at[dst_v.at[j]], ssems[b],
                             add=True)
            for k in range(CHUNK // 16):
                idx = dst_v[j, pl.ds(k * 16, 16)]
                plsc.addupdate_scatter(
                    cnt_tile, [idx], jnp.full((16,), 1.0, jnp.float32))
        return carry

    lax.fori_loop(0, outer, body, 0)

    for b in range(NBUF):
        j = CH_PER_TILE - NBUF + b
        pltpu.make_async_copy(fbufs[b], acc_sh.at[dst_v.at[j]],
                              ssems[b]).wait()
    plsc.subcore_barrier()

    pltpu.sync_copy(acc_sh.at[pl.ds(base, ROWS_PER_TILE)],
                    acc_out.at[cid, pl.ds(base, ROWS_PER_TILE)])
    pltpu.sync_copy(cnt_tile, cnt_out.at[wid])


_sc_segment_sum = functools.partial(
    pl.kernel,
    out_type=(
        jax.ShapeDtypeStruct((NC, ACC_ROWS, HIDDEN), jnp.float32),
        jax.ShapeDtypeStruct((NW, ACC_ROWS), jnp.float32),
    ),
    mesh=plsc.VectorSubcoreMesh(core_axis_name="c", subcore_axis_name="s"),
    compiler_params=pltpu.CompilerParams(
        use_tc_tiling_on_sc=False, needs_layout_passes=False),
    scratch_types=[
        pltpu.VMEM((CH_PER_TILE, CHUNK), jnp.int32),
        pltpu.VMEM((CH_PER_TILE, CHUNK), jnp.int32),
        pltpu.VMEM((CHUNK, ZPK), jnp.int32),
        pltpu.VMEM((CHUNK, ZPK), jnp.int32),
        pltpu.VMEM((CHUNK, HIDDEN), jnp.float32),
        pltpu.VMEM((CHUNK, HIDDEN), jnp.float32),
        pltpu.VMEM((ACC_ROWS,), jnp.float32),
        pltpu.VMEM_SHARED((Z_ROWS, ZPK), jnp.int32),
        pltpu.VMEM_SHARED((ACC_ROWS, HIDDEN), jnp.float32),
        pltpu.SemaphoreType.DMA,
        pltpu.SemaphoreType.DMA,
        pltpu.SemaphoreType.DMA,
        pltpu.SemaphoreType.DMA,
    ],
)(_sc_body)


@jax.jit
def kernel(x, edge_index, W_l, b_l, W_r, W_head, b_head):
    src = edge_index[0].astype(jnp.int32)
    dst = edge_index[1].astype(jnp.int32)
    pad = E_PAD - N_EDGES
    src_p = jnp.concatenate([src, jnp.zeros((pad,), jnp.int32)])
    dst_p = jnp.concatenate([dst, jnp.full((pad,), N_NODES, jnp.int32)])
    src_r = src_p.reshape(NW, CH_PER_TILE, CHUNK)
    dst_r = dst_p.reshape(NW, CH_PER_TILE, CHUNK)

    w_perm = W_l[jnp.asarray(_INV_SIGMA)]
    z16 = pl.pallas_call(
        _proj_body,
        out_shape=jax.ShapeDtypeStruct((Z_ROWS, HIDDEN), jnp.bfloat16),
    )(x, w_perm)
    z_packed = lax.bitcast_convert_type(
        z16.reshape(Z_ROWS, ZPK, 2), jnp.int32)

    zrow = jnp.zeros((ROWS_PER_TILE, HIDDEN), jnp.float32)
    zcnt = jnp.zeros((ACC_ROWS,), jnp.float32)
    acc, cnt = _sc_segment_sum(z_packed, src_r, dst_r, zrow, zcnt)

    y = pl.pallas_call(
        _post_body,
        in_specs=[
            pl.BlockSpec(memory_space=pltpu.VMEM),
            pl.BlockSpec(memory_space=pltpu.VMEM),
            pl.BlockSpec(memory_space=pltpu.VMEM),
            pl.BlockSpec(memory_space=pltpu.VMEM),
            pl.BlockSpec(memory_space=pltpu.VMEM),
            pl.BlockSpec(memory_space=pltpu.VMEM),
            pl.BlockSpec(memory_space=pltpu.SMEM),
        ],
        out_shape=jax.ShapeDtypeStruct((N_NODES, 1), jnp.float32),
    )(acc, cnt, x, W_r, b_l.reshape(1, HIDDEN), W_head.reshape(HIDDEN, 1),
      b_head.reshape(1, 1))
    return jnp.squeeze(y, axis=-1)
